# initial kernel scaffold (unmeasured)
import jax
import jax.numpy as jnp
from jax import lax
from jax.experimental import pallas as pl
from jax.experimental.pallas import tpu as pltpu

NZ = 4
M, N = 8192, 2048
N_DIR = 2
N_CHUNK = 4
ROWS_PER_DIR = M // N_DIR
ROWS_PER_CHUNK = ROWS_PER_DIR // N_CHUNK
SEG = ROWS_PER_CHUNK // NZ
EPS = 1e-6


def kernel(partial, resid, gamma):
    def body(
        partial_ref,
        resid_ref,
        gamma_ref,
        out_ref,
        rs_send,
        rs_recv,
        ag,
        pstage,
        rstage,
        outstage,
        rs_send_sems,
        rs_recv_sems,
        ag_send_sems,
        ag_recv_sems,
        pstage_sems,
        rstage_sems,
        out_sems,
    ):
        xi = lax.axis_index("x")
        yi = lax.axis_index("y")
        zi = lax.axis_index("z")
        nbr = [(zi + 1) % NZ, (zi + NZ - 1) % NZ]

        barrier = pltpu.get_barrier_semaphore()
        for d in range(N_DIR):
            pl.semaphore_signal(
                barrier,
                inc=1,
                device_id=(xi, yi, nbr[d]),
                device_id_type=pl.DeviceIdType.MESH,
            )
        pl.semaphore_wait(barrier, 2)

        def seg_rows(d, c, seg):
            return d * ROWS_PER_DIR + c * ROWS_PER_CHUNK + seg * SEG

        def seg_send(d, k):
            return (zi + (NZ - k) % NZ) % NZ if d == 0 else (zi + k) % NZ

        def seg_recv(d, k):
            return (zi + NZ - k - 1) % NZ if d == 0 else (zi + k + 1) % NZ

        def owned(d):
            return (zi + 1) % NZ if d == 0 else (zi + NZ - 1) % NZ

        def ag_recv_seg(d, k):
            return (zi + (NZ - k) % NZ) % NZ if d == 0 else (zi + k) % NZ

        def start_pstage(d, c, seg, slot):
            cp = pltpu.make_async_copy(
                partial_ref.at[0, pl.ds(seg_rows(d, c, seg), SEG), :],
                pstage.at[d, slot],
                pstage_sems.at[d, slot],
            )
            cp.start()
            return cp

        out_pending = {}

        def wait_out(d, slot):
            cp = out_pending.pop((d, slot), None)
            if cp is not None:
                cp.wait()

        def start_out(d, slot, row):
            cp = pltpu.make_async_copy(
                outstage.at[d, slot],
                out_ref.at[pl.ds(row, SEG), :],
                out_sems.at[d, slot],
            )
            cp.start()
            out_pending[(d, slot)] = cp

        for c in range(N_CHUNK):
            ps = [None, None]
            rstage_cp = [None, None]
            for d in range(N_DIR):
                ps[d] = start_pstage(d, c, seg_send(d, 0), 0)
                rcp = pltpu.make_async_copy(
                    resid_ref.at[pl.ds(seg_rows(d, c, owned(d)), SEG), :],
                    rstage.at[d],
                    rstage_sems.at[d],
                )
                rcp.start()
                rstage_cp[d] = rcp
            for d in range(N_DIR):
                ps[d].wait()
                rs_send[d, 0, :, :] = pstage[d, 0].astype(jnp.bfloat16)
                ps[d] = start_pstage(d, c, seg_recv(d, 0), 1)

            for k in range(NZ - 1):
                rdmas = []
                for d in range(N_DIR):
                    r = pltpu.make_async_remote_copy(
                        src_ref=rs_send.at[d, k % 2],
                        dst_ref=rs_recv.at[d, k],
                        send_sem=rs_send_sems.at[d, k],
                        recv_sem=rs_recv_sems.at[d, k],
                        device_id=(xi, yi, nbr[d]),
                        device_id_type=pl.DeviceIdType.MESH,
                    )
                    r.start()
                    rdmas.append(r)
                for d in range(N_DIR):
                    nxt = None
                    if k < NZ - 2:
                        nxt = start_pstage(d, c, seg_recv(d, k + 1), k % 2)
                    ps[d].wait()
                    rdmas[d].wait()
                    cur = (k + 1) % 2
                    if k < NZ - 2:
                        acc = rs_recv[d, k].astype(jnp.float32) + pstage[d, cur]
                        rs_send[d, (k + 1) % 2, :, :] = acc.astype(jnp.bfloat16)
                        ps[d] = nxt
                    else:
                        rstage_cp[d].wait()
                        y = (
                            rs_recv[d, k].astype(jnp.float32)
                            + pstage[d, cur]
                            + rstage[d]
                        )
                        ms = jnp.mean(y * y, axis=-1, keepdims=True)
                        o = y * lax.rsqrt(ms + EPS) * gamma_ref[...]
                        wait_out(d, 0)
                        outstage[d, 0, :, :] = o
                        start_out(d, 0, seg_rows(d, c, owned(d)))
                        ag[d, 0, :, :] = o.astype(jnp.bfloat16)

            for k in range(NZ - 1):
                rdmas = []
                for d in range(N_DIR):
                    r = pltpu.make_async_remote_copy(
                        src_ref=ag.at[d, k],
                        dst_ref=ag.at[d, k + 1],
                        send_sem=ag_send_sems.at[d, k],
                        recv_sem=ag_recv_sems.at[d, k],
                        device_id=(xi, yi, nbr[d]),
                        device_id_type=pl.DeviceIdType.MESH,
                    )
                    r.start()
                    rdmas.append(r)
                for d in range(N_DIR):
                    rdmas[d].wait()
                    slot = 1 - (k % 2)
                    wait_out(d, slot)
                    outstage[d, slot, :, :] = ag[d, k + 1].astype(jnp.float32)
                    start_out(d, slot, seg_rows(d, c, ag_recv_seg(d, k)))

        for d, slot in list(out_pending):
            wait_out(d, slot)

    return pl.pallas_call(
        body,
        out_shape=jax.ShapeDtypeStruct((M, N), jnp.float32),
        in_specs=[
            pl.BlockSpec(memory_space=pltpu.ANY),
            pl.BlockSpec(memory_space=pltpu.ANY),
            pl.BlockSpec(memory_space=pltpu.VMEM),
        ],
        out_specs=pl.BlockSpec(memory_space=pltpu.ANY),
        scratch_shapes=[
            pltpu.VMEM((N_DIR, 2, SEG, N), jnp.bfloat16),
            pltpu.VMEM((N_DIR, NZ - 1, SEG, N), jnp.bfloat16),
            pltpu.VMEM((N_DIR, NZ, SEG, N), jnp.bfloat16),
            pltpu.VMEM((N_DIR, 2, SEG, N), jnp.float32),
            pltpu.VMEM((N_DIR, SEG, N), jnp.float32),
            pltpu.VMEM((N_DIR, 2, SEG, N), jnp.float32),
            pltpu.SemaphoreType.DMA((N_DIR, NZ - 1)),
            pltpu.SemaphoreType.DMA((N_DIR, NZ - 1)),
            pltpu.SemaphoreType.DMA((N_DIR, NZ - 1)),
            pltpu.SemaphoreType.DMA((N_DIR, NZ - 1)),
            pltpu.SemaphoreType.DMA((N_DIR, 2)),
            pltpu.SemaphoreType.DMA((N_DIR,)),
            pltpu.SemaphoreType.DMA((N_DIR, 2)),
        ],
        compiler_params=pltpu.CompilerParams(collective_id=0),
    )(partial, resid, gamma)


# baseline (device time: 656952 ns/iter reference)
import jax
import jax.numpy as jnp
from jax import lax
from jax.experimental import pallas as pl
from jax.experimental.pallas import tpu as pltpu

NZ = 4
M, N = 8192, 2048
N_DIR = 2
N_CHUNK = 4
ROWS_PER_DIR = M // N_DIR
ROWS_PER_CHUNK = ROWS_PER_DIR // N_CHUNK
SEG = ROWS_PER_CHUNK // NZ
EPS = 1e-6


def kernel(partial, resid, gamma):
    def body(
        partial_ref,
        resid_ref,
        gamma_ref,
        out_ref,
        rs_send,
        rs_recv,
        ag,
        pstage,
        rstage,
        outstage,
        rs_send_sems,
        rs_recv_sems,
        ag_send_sems,
        ag_recv_sems,
        pstage_sems,
        rstage_sems,
        out_sems,
    ):
        xi = lax.axis_index("x")
        yi = lax.axis_index("y")
        zi = lax.axis_index("z")
        nbr = [(zi + 1) % NZ, (zi + NZ - 1) % NZ]

        barrier = pltpu.get_barrier_semaphore()
        for d in range(N_DIR):
            pl.semaphore_signal(
                barrier,
                inc=1,
                device_id=(xi, yi, nbr[d]),
                device_id_type=pl.DeviceIdType.MESH,
            )
        pl.semaphore_wait(barrier, 2)

        def seg_rows(d, c, seg):
            return d * ROWS_PER_DIR + c * ROWS_PER_CHUNK + seg * SEG

        def seg_send(d, k):
            return (zi + (NZ - k) % NZ) % NZ if d == 0 else (zi + k) % NZ

        def seg_recv(d, k):
            return (zi + NZ - k - 1) % NZ if d == 0 else (zi + k + 1) % NZ

        def owned(d):
            return (zi + 1) % NZ if d == 0 else (zi + NZ - 1) % NZ

        def ag_recv_seg(d, k):
            return (zi + (NZ - k) % NZ) % NZ if d == 0 else (zi + k) % NZ

        def start_pstage(d, c, seg, slot):
            cp = pltpu.make_async_copy(
                partial_ref.at[0, pl.ds(seg_rows(d, c, seg), SEG), :],
                pstage.at[d, slot],
                pstage_sems.at[d, slot],
            )
            cp.start()
            return cp

        out_pending = {}

        def wait_out(d, slot):
            cp = out_pending.pop((d, slot), None)
            if cp is not None:
                cp.wait()

        def start_out(d, slot, row):
            cp = pltpu.make_async_copy(
                outstage.at[d, slot],
                out_ref.at[pl.ds(row, SEG), :],
                out_sems.at[d, slot],
            )
            cp.start()
            out_pending[(d, slot)] = cp

        for c in range(N_CHUNK):
            ps = [None, None]
            rstage_cp = [None, None]
            for d in range(N_DIR):
                ps[d] = start_pstage(d, c, seg_send(d, 0), 0)
                rcp = pltpu.make_async_copy(
                    resid_ref.at[pl.ds(seg_rows(d, c, owned(d)), SEG), :],
                    rstage.at[d],
                    rstage_sems.at[d],
                )
                rcp.start()
                rstage_cp[d] = rcp
            for d in range(N_DIR):
                ps[d].wait()
                rs_send[d, 0, :, :] = pstage[d, 0].astype(jnp.bfloat16)
                ps[d] = start_pstage(d, c, seg_recv(d, 0), 1)

            for k in range(NZ - 1):
                rdmas = []
                for d in range(N_DIR):
                    r = pltpu.make_async_remote_copy(
                        src_ref=rs_send.at[d, k % 2],
                        dst_ref=rs_recv.at[d, k],
                        send_sem=rs_send_sems.at[d, k],
                        recv_sem=rs_recv_sems.at[d, k],
                        device_id=(xi, yi, nbr[d]),
                        device_id_type=pl.DeviceIdType.MESH,
                    )
                    r.start()
                    rdmas.append(r)
                for d in range(N_DIR):
                    nxt = None
                    if k < NZ - 2:
                        nxt = start_pstage(d, c, seg_recv(d, k + 1), k % 2)
                    ps[d].wait()
                    rdmas[d].wait()
                    cur = (k + 1) % 2
                    if k < NZ - 2:
                        acc = rs_recv[d, k].astype(jnp.float32) + pstage[d, cur]
                        rs_send[d, (k + 1) % 2, :, :] = acc.astype(jnp.bfloat16)
                        ps[d] = nxt
                    else:
                        rstage_cp[d].wait()
                        y = (
                            rs_recv[d, k].astype(jnp.float32)
                            + pstage[d, cur]
                            + rstage[d]
                        )
                        ms = jnp.mean(y * y, axis=-1, keepdims=True)
                        o = y * lax.rsqrt(ms + EPS) * gamma_ref[...]
                        wait_out(d, 0)
                        outstage[d, 0, :, :] = o
                        start_out(d, 0, seg_rows(d, c, owned(d)))
                        ag[d, 0, :, :] = o.astype(jnp.bfloat16)

            for k in range(NZ - 1):
                rdmas = []
                for d in range(N_DIR):
                    r = pltpu.make_async_remote_copy(
                        src_ref=ag.at[d, k],
                        dst_ref=ag.at[d, k + 1],
                        send_sem=ag_send_sems.at[d, k],
                        recv_sem=ag_recv_sems.at[d, k],
                        device_id=(xi, yi, nbr[d]),
                        device_id_type=pl.DeviceIdType.MESH,
                    )
                    r.start()
                    rdmas.append(r)
                for d in range(N_DIR):
                    rdmas[d].wait()
                    slot = 1 - (k % 2)
                    wait_out(d, slot)
                    outstage[d, slot, :, :] = ag[d, k + 1].astype(jnp.float32)
                    start_out(d, slot, seg_rows(d, c, ag_recv_seg(d, k)))

        for d, slot in list(out_pending):
            wait_out(d, slot)

    return pl.pallas_call(
        body,
        out_shape=jax.ShapeDtypeStruct((M, N), jnp.float32),
        in_specs=[
            pl.BlockSpec(memory_space=pl.ANY),
            pl.BlockSpec(memory_space=pl.ANY),
            pl.BlockSpec(memory_space=pltpu.MemorySpace.VMEM),
        ],
        out_specs=pl.BlockSpec(memory_space=pl.ANY),
        scratch_shapes=[
            pltpu.VMEM((N_DIR, 2, SEG, N), jnp.bfloat16),
            pltpu.VMEM((N_DIR, NZ - 1, SEG, N), jnp.bfloat16),
            pltpu.VMEM((N_DIR, NZ, SEG, N), jnp.bfloat16),
            pltpu.VMEM((N_DIR, 2, SEG, N), jnp.float32),
            pltpu.VMEM((N_DIR, SEG, N), jnp.float32),
            pltpu.VMEM((N_DIR, 2, SEG, N), jnp.float32),
            pltpu.SemaphoreType.DMA((N_DIR, NZ - 1)),
            pltpu.SemaphoreType.DMA((N_DIR, NZ - 1)),
            pltpu.SemaphoreType.DMA((N_DIR, NZ - 1)),
            pltpu.SemaphoreType.DMA((N_DIR, NZ - 1)),
            pltpu.SemaphoreType.DMA((N_DIR, 2)),
            pltpu.SemaphoreType.DMA((N_DIR,)),
            pltpu.SemaphoreType.DMA((N_DIR, 2)),
        ],
        compiler_params=pltpu.CompilerParams(
            collective_id=0,
            vmem_limit_bytes=56 * 1024 * 1024,
        ),
    )(partial, resid, gamma)
